# baseline (speedup 1.0000x reference)
import functools

import jax
import jax.numpy as jnp
from jax import lax
from jax.experimental import pallas as pl
from jax.experimental.pallas import tpu as pltpu
from jax.experimental.pallas import tpu_sc as plsc

N = 10000
E = 320000
G = 64
K = 30
NC = 2
NS = 16
NW = NC * NS
EP = E // NW
CH = 80
NCHUNK = EP // CH
NACC = 10240
ZR = NACC // NS
NEG = -1e30
NPAD = N + 16
BPOOL = G * 32


NBUF = 5


def _make_sc_seg(F, gather):
    mesh = plsc.VectorSubcoreMesh(core_axis_name="c", subcore_axis_name="s")
    scratch = [
        pltpu.VMEM((EP,), jnp.int32),
        pltpu.VMEM((EP,), jnp.int32),
        pltpu.VMEM((1, CH), jnp.int32),
        pltpu.VMEM((NBUF, CH, F), jnp.float32),
        pltpu.VMEM((ZR, F), jnp.float32),
        pltpu.VMEM_SHARED((NACC, F), jnp.float32),
    ] + [pltpu.SemaphoreType.DMA] * NBUF
    out_type = jax.ShapeDtypeStruct((NC, NACC, F), jnp.float32)

    def body(*refs):
        if gather:
            (table, src_h, dst_h, out, srca, dsta, dstv, rows, zero_v,
             acc, *gsems) = refs
        else:
            (src_h, dst_h, out, srca, dsta, dstv, rows, zero_v,
             acc, *gsems) = refs
        c = lax.axis_index("c")
        s = lax.axis_index("s")
        wid = s * NC + c
        ebase = wid * EP

        pltpu.sync_copy(dst_h.at[pl.ds(ebase, EP)], dsta)
        if gather:
            pltpu.sync_copy(src_h.at[pl.ds(ebase, EP)], srca)

        zvec = jnp.zeros((16,), jnp.float32)

        def zrow(i, carry):
            for j in range(F // 16):
                zero_v[i, pl.ds(j * 16, 16)] = zvec
            return carry

        lax.fori_loop(0, ZR, zrow, 0)
        pltpu.sync_copy(zero_v, acc.at[pl.ds(s * ZR, ZR)])
        if gather:
            for b in range(NBUF):
                pltpu.async_copy(table.at[srca.at[pl.ds(b * CH, CH)]],
                                 rows.at[b], gsems[b])
        else:
            ovec = jnp.ones((16,), jnp.float32)

            def orow(i, carry):
                for j in range(F // 16):
                    rows[0, i, pl.ds(j * 16, 16)] = ovec
                return carry

            lax.fori_loop(0, CH, orow, 0)
        plsc.subcore_barrier()

        def group(gidx, carry):
            for b in range(NBUF):
                i = gidx * NBUF + b
                for j in range(CH // 16):
                    dstv[0, pl.ds(j * 16, 16)] = dsta[pl.ds(i * CH + j * 16, 16)]
                if gather:
                    pltpu.make_async_copy(table.at[pl.ds(0, CH)], rows.at[b],
                                          gsems[b]).wait()
                    pltpu.sync_copy(rows.at[b], acc.at[dstv.at[0]], add=True)

                    @pl.when(i + NBUF < NCHUNK)
                    def _():
                        pltpu.async_copy(
                            table.at[srca.at[pl.ds((i + NBUF) * CH, CH)]],
                            rows.at[b], gsems[b])
                else:
                    pltpu.sync_copy(rows.at[0], acc.at[dstv.at[0]], add=True)
            return carry

        lax.fori_loop(0, NCHUNK // NBUF, group, 0)
        plsc.subcore_barrier()
        pltpu.sync_copy(acc.at[pl.ds(s * ZR, ZR)], out.at[c, pl.ds(s * ZR, ZR)])

    return pl.kernel(body, out_type=out_type, mesh=mesh, scratch_types=scratch,
                     compiler_params=pltpu.CompilerParams(use_tc_tiling_on_sc=False))


def _make_sc_pool_gather():
    mesh = plsc.VectorSubcoreMesh(core_axis_name="c", subcore_axis_name="s")
    per_w = BPOOL // NW
    scratch = [
        pltpu.VMEM((per_w,), jnp.int32),
        pltpu.VMEM((per_w, 112), jnp.float32),
        pltpu.SemaphoreType.DMA,
    ]
    out_type = jax.ShapeDtypeStruct((BPOOL, 112), jnp.float32)

    def body(table, idx_h, out, idx_v, rows_v, sem):
        c = lax.axis_index("c")
        s = lax.axis_index("s")
        wid = s * NC + c
        base = wid * per_w
        pltpu.sync_copy(idx_h.at[pl.ds(base, per_w)], idx_v)
        pltpu.async_copy(table.at[idx_v], rows_v, sem).wait()
        pltpu.sync_copy(rows_v, out.at[pl.ds(base, per_w)])

    return pl.kernel(body, out_type=out_type, mesh=mesh, scratch_types=scratch,
                     compiler_params=pltpu.CompilerParams(use_tc_tiling_on_sc=False))


def _t1_body(cnt_ref, x_ref, w1_ref, dis_ref, g1_ref):
    deg = cnt_ref[0, :N, 0:1] + cnt_ref[1, :N, 0:1] + 1.0
    dis = lax.rsqrt(deg)
    m = jnp.dot(x_ref[...], w1_ref[...], preferred_element_type=jnp.float32)
    dis_ref[...] = dis
    g1_ref[...] = dis * m


def _t2_body(ap_ref, g_ref, dis_ref, b_ref, w_ref, h_ref, gn_ref):
    dis = dis_ref[...]
    h = jnp.tanh(dis * (ap_ref[0, :N, :] + ap_ref[1, :N, :] + g_ref[...]) + b_ref[...])
    h_ref[...] = h
    gn_ref[...] = dis * jnp.dot(h, w_ref[...], preferred_element_type=jnp.float32)


def _t5_body(bp_ref, u_ref, dis_ref, w4_ref, b4_ref, h1_ref, h2_ref, h3_ref,
             batch_ref, hcat_ref, idx_ref):
    dis = dis_ref[...]
    sfin = dis * (bp_ref[0, :N, :] + bp_ref[1, :N, :] + u_ref[...])
    h4 = jnp.tanh(jnp.dot(sfin, w4_ref[...], preferred_element_type=jnp.float32)
                  + b4_ref[...])
    hc = jnp.concatenate(
        [h1_ref[...], h2_ref[...], h3_ref[...], h4,
         jnp.zeros((N, 15), jnp.float32)], axis=1)
    hcat_ref[...] = jnp.concatenate(
        [hc, jnp.zeros((NPAD - N, 112), jnp.float32)], axis=0)

    gid = lax.broadcasted_iota(jnp.int32, (1, G), 1)
    rowi = lax.broadcasted_iota(jnp.int32, (N, 1), 0)
    krow = lax.broadcasted_iota(jnp.int32, (32, 1), 0)
    m0 = jnp.where(batch_ref[...] == gid, h4, NEG)

    def step(k, carry):
        vt, li, idxacc = carry
        avail = (m0 < vt) | ((m0 == vt) & (rowi > li))
        cand = jnp.where(avail, m0, NEG)
        mx = jnp.max(cand, axis=0, keepdims=True)
        am = jnp.min(jnp.where(cand == mx, rowi, N), axis=0, keepdims=True)
        am = jnp.where(mx > -1e29, am, N)
        idxacc = jnp.where(krow == k, am, idxacc)
        return mx, am, idxacc

    _, _, idxacc = lax.fori_loop(
        0, K, step, (jnp.full((1, G), 1e30, jnp.float32),
                     jnp.full((1, G), -1, jnp.int32),
                     jnp.full((32, G), N, jnp.int32)))
    idx_ref[...] = idxacc


def _t6_body(pooled_ref, w1c_ref, b1c_ref, wc2_ref, b2c_ref,
             l1w_ref, l1b_ref, l2w_ref, l2b_ref, out_ref):
    pr = pooled_ref[...]
    c1 = jnp.maximum(
        jnp.dot(pr, w1c_ref[...], preferred_element_type=jnp.float32)
        + b1c_ref[...], 0.0)
    c3 = c1.reshape(G, 16, 2, 16)
    p = jnp.max(c3, axis=2)[:, :15, :]
    c2s = []
    for j in range(11):
        win = jnp.concatenate([p[:, j + t, :] for t in range(5)], axis=1)
        c2s.append(jnp.maximum(
            jnp.dot(win, wc2_ref[...], preferred_element_type=jnp.float32)
            + b2c_ref[...], 0.0))
    f = jnp.concatenate([c[:, :, None] for c in c2s], axis=2).reshape(G, 352)
    f = jnp.maximum(
        jnp.dot(f, l1w_ref[...], preferred_element_type=jnp.float32)
        + l1b_ref[...], 0.0)
    z = jnp.dot(f, l2w_ref[...], preferred_element_type=jnp.float32) + l2b_ref[...]
    out_ref[...] = jax.nn.sigmoid(z)


def _tc(body, out_shapes):
    return pl.pallas_call(
        body, out_shape=out_shapes,
        compiler_params=pltpu.CompilerParams(vmem_limit_bytes=100 * 1024 * 1024))


_sc_count = _make_sc_seg(16, gather=False)
_sc_seg32 = _make_sc_seg(32, gather=True)
_sc_pool = _make_sc_pool_gather()

_t1 = _tc(_t1_body, [jax.ShapeDtypeStruct((N, 1), jnp.float32),
                     jax.ShapeDtypeStruct((N, 32), jnp.float32)])
_t2 = _tc(_t2_body, [jax.ShapeDtypeStruct((N, 32), jnp.float32),
                     jax.ShapeDtypeStruct((N, 32), jnp.float32)])
_t5 = _tc(_t5_body, [jax.ShapeDtypeStruct((NPAD, 112), jnp.float32),
                     jax.ShapeDtypeStruct((32, G), jnp.int32)])
_t6 = _tc(_t6_body, [jax.ShapeDtypeStruct((G, 1), jnp.float32)])


def kernel(x, edge_index, batch, W1, b1, W2, b2, W3, b3, W4, b4,
           conv1_w, conv1_b, conv2_w, conv2_b, lin1_w, lin1_b, lin2_w, lin2_b):
    src = edge_index[0]
    dst = edge_index[1]

    cnt = _sc_count(src, dst)
    dis, g1 = _t1(cnt, x, W1)
    a1 = _sc_seg32(g1, src, dst)
    h1, g2 = _t2(a1, g1, dis, b1.reshape(1, 32), W2)
    a2 = _sc_seg32(g2, src, dst)
    h2, g3 = _t2(a2, g2, dis, b2.reshape(1, 32), W3)
    a3 = _sc_seg32(g3, src, dst)
    h3, u = _t2(a3, g3, dis, b3.reshape(1, 32), jnp.eye(32, dtype=jnp.float32))
    bb = _sc_seg32(u, src, dst)
    hcat, idxacc = _t5(bb, u, dis, W4, b4.reshape(1, 1), h1, h2, h3,
                       batch.reshape(N, 1))
    idx_flat = idxacc.T.reshape(BPOOL)
    pooled = _sc_pool(hcat, idx_flat)

    w1c = jnp.concatenate(
        [conv1_w[:, 0, :].T, jnp.zeros((15, 16), jnp.float32)], axis=0)
    wc2 = conv2_w.transpose(2, 1, 0).reshape(80, 32)
    [out] = _t6(pooled, w1c, conv1_b.reshape(1, 16), wc2,
                conv2_b.reshape(1, 32), lin1_w, lin1_b.reshape(1, 128),
                lin2_w, lin2_b.reshape(1, 1))
    return out


# speedup vs baseline: 1.2744x; 1.2744x over previous
import functools

import jax
import jax.numpy as jnp
from jax import lax
from jax.experimental import pallas as pl
from jax.experimental.pallas import tpu as pltpu
from jax.experimental.pallas import tpu_sc as plsc

N = 10000
E = 320000
G = 64
K = 30
NC = 2
NS = 16
NW = NC * NS
EP = E // NW
CH = 128
NCHF = EP // CH
TAIL = EP - NCHF * CH
NACC = 10240
ZR = NACC // NS
NEG = -1e30
NPAD = N + 16
BPOOL = G * 32


NBUF = 6


def _make_sc_seg(F, gather):
    mesh = plsc.VectorSubcoreMesh(core_axis_name="c", subcore_axis_name="s")
    scratch = [
        pltpu.VMEM((EP,), jnp.int32),
        pltpu.VMEM((EP,), jnp.int32),
        pltpu.VMEM((1, CH), jnp.int32),
        pltpu.VMEM((1, TAIL), jnp.int32),
        pltpu.VMEM((NBUF, CH, F), jnp.float32),
        pltpu.VMEM((TAIL, F), jnp.float32),
        pltpu.VMEM((ZR, F), jnp.float32),
        pltpu.VMEM_SHARED((NACC, F), jnp.float32),
    ] + [pltpu.SemaphoreType.DMA] * (NBUF + 1)
    out_type = jax.ShapeDtypeStruct((NC, NACC, F), jnp.float32)

    def body(*refs):
        if gather:
            (table, src_h, dst_h, out, srca, dsta, dstv, dstv2, rows, trows,
             zero_v, acc, *gsems) = refs
        else:
            (src_h, dst_h, out, srca, dsta, dstv, dstv2, rows, trows,
             zero_v, acc, *gsems) = refs
        c = lax.axis_index("c")
        s = lax.axis_index("s")
        wid = s * NC + c
        ebase = wid * EP

        pltpu.sync_copy(dst_h.at[pl.ds(ebase, EP)], dsta)
        if gather:
            pltpu.sync_copy(src_h.at[pl.ds(ebase, EP)], srca)

        zvec = jnp.zeros((16,), jnp.float32)

        def zrow(i, carry):
            for j in range(F // 16):
                zero_v[i, pl.ds(j * 16, 16)] = zvec
            return carry

        lax.fori_loop(0, ZR, zrow, 0)
        pltpu.sync_copy(zero_v, acc.at[pl.ds(s * ZR, ZR)])
        if gather:
            for b in range(NBUF):
                pltpu.async_copy(table.at[srca.at[pl.ds(b * CH, CH)]],
                                 rows.at[b], gsems[b])
            pltpu.async_copy(table.at[srca.at[pl.ds(NCHF * CH, TAIL)]],
                             trows, gsems[NBUF])
        else:
            ovec = jnp.ones((16,), jnp.float32)

            def orow(i, carry):
                for j in range(F // 16):
                    rows[0, i, pl.ds(j * 16, 16)] = ovec
                return carry

            lax.fori_loop(0, CH, orow, 0)

            def otrow(i, carry):
                for j in range(F // 16):
                    trows[i, pl.ds(j * 16, 16)] = ovec
                return carry

            lax.fori_loop(0, TAIL, otrow, 0)
        plsc.subcore_barrier()

        def group(gidx, carry):
            for b in range(NBUF):
                i = gidx * NBUF + b
                for j in range(CH // 16):
                    dstv[0, pl.ds(j * 16, 16)] = dsta[pl.ds(i * CH + j * 16, 16)]
                if gather:
                    pltpu.make_async_copy(table.at[pl.ds(0, CH)], rows.at[b],
                                          gsems[b]).wait()
                    pltpu.sync_copy(rows.at[b], acc.at[dstv.at[0]], add=True)

                    @pl.when(i + NBUF < NCHF)
                    def _():
                        pltpu.async_copy(
                            table.at[srca.at[pl.ds((i + NBUF) * CH, CH)]],
                            rows.at[b], gsems[b])
                else:
                    pltpu.sync_copy(rows.at[0], acc.at[dstv.at[0]], add=True)
            return carry

        lax.fori_loop(0, NCHF // NBUF, group, 0)
        for j in range(TAIL // 16):
            dstv2[0, pl.ds(j * 16, 16)] = dsta[pl.ds(NCHF * CH + j * 16, 16)]
        if gather:
            pltpu.make_async_copy(table.at[pl.ds(0, TAIL)], trows,
                                  gsems[NBUF]).wait()
        pltpu.sync_copy(trows, acc.at[dstv2.at[0]], add=True)
        plsc.subcore_barrier()
        pltpu.sync_copy(acc.at[pl.ds(s * ZR, ZR)], out.at[c, pl.ds(s * ZR, ZR)])

    return pl.kernel(body, out_type=out_type, mesh=mesh, scratch_types=scratch,
                     compiler_params=pltpu.CompilerParams(use_tc_tiling_on_sc=False))


def _make_sc_pool_topk():
    mesh = plsc.VectorSubcoreMesh(core_axis_name="c", subcore_axis_name="s")
    per_w = BPOOL // NW
    scratch = [
        pltpu.VMEM((NPAD,), jnp.float32),
        pltpu.VMEM((G + 16,), jnp.int32),
        pltpu.VMEM((G + 16,), jnp.int32),
        pltpu.VMEM((1, per_w), jnp.int32),
        pltpu.VMEM((per_w, 112), jnp.float32),
        pltpu.SemaphoreType.DMA,
    ]
    out_type = jax.ShapeDtypeStruct((BPOOL, 112), jnp.float32)
    LOW = -3e38

    def body(table, key_h, cnt_h, sta_h, out, key_v, cnt_v, sta_v,
             idx_v, rows_v, sem):
        c = lax.axis_index("c")
        s = lax.axis_index("s")
        wid = s * NC + c
        pltpu.sync_copy(key_h, key_v)
        pltpu.sync_copy(cnt_h, cnt_v.at[pl.ds(0, G)])
        pltpu.sync_copy(sta_h, sta_v.at[pl.ds(0, G)])
        lane = lax.iota(jnp.int32, 16)
        sent = jnp.full((16,), N, jnp.int32)
        for h in range(per_w // 16):
            idx_v[0, pl.ds(h * 16, 16)] = sent
        for t in range(2):
            g = wid * 2 + t
            cnt = cnt_v[pl.ds(g, 16)][0]
            start = sta_v[pl.ds(g, 16)][0]
            nv = (cnt + 15) >> 4

            def select(k, kcarry):
                vt, li = kcarry

                def scan(j, carry):
                    bv, bp = carry
                    v = key_v[pl.ds(start + j * 16, 16)]
                    pos = lane + j * 16
                    ok = ((pos < cnt)
                          & ((v < vt) | ((v == vt) & (pos > li))))
                    vm = jnp.where(ok, v, LOW)
                    m = jnp.max(vm)
                    p = jnp.min(jnp.where(vm == m, pos, N))
                    better = m > bv
                    return (jnp.where(better, m, bv),
                            jnp.where(better, p, bp))

                bv, bp = lax.fori_loop(0, nv, scan,
                                       (jnp.float32(LOW), jnp.int32(N)))
                gi = jnp.where(bv > jnp.float32(-1e30), start + bp, N)
                slot = t * 32 + k
                hb = (slot >> 4) << 4
                lane_in = slot & 15
                cur = idx_v[0, pl.ds(hb, 16)]
                idx_v[0, pl.ds(hb, 16)] = jnp.where(lane == lane_in, gi, cur)
                return bv, bp

            lax.fori_loop(0, K, select, (jnp.float32(3e38), jnp.int32(-1)))
        pltpu.async_copy(table.at[idx_v.at[0]], rows_v, sem).wait()
        pltpu.sync_copy(rows_v, out.at[pl.ds(wid * per_w, per_w)])

    return pl.kernel(body, out_type=out_type, mesh=mesh, scratch_types=scratch,
                     compiler_params=pltpu.CompilerParams(
                         use_tc_tiling_on_sc=False, needs_layout_passes=False))


def _t1_body(cnt_ref, x_ref, w1_ref, dis_ref, g1_ref):
    deg = cnt_ref[0, :N, 0:1] + cnt_ref[1, :N, 0:1] + 1.0
    dis = lax.rsqrt(deg)
    m = jnp.dot(x_ref[...], w1_ref[...], preferred_element_type=jnp.float32)
    dis_ref[...] = dis
    g1_ref[...] = dis * m


def _t2_body(ap_ref, g_ref, dis_ref, b_ref, w_ref, h_ref, gn_ref):
    dis = dis_ref[...]
    h = jnp.tanh(dis * (ap_ref[0, :N, :] + ap_ref[1, :N, :] + g_ref[...]) + b_ref[...])
    h_ref[...] = h
    gn_ref[...] = dis * jnp.dot(h, w_ref[...], preferred_element_type=jnp.float32)


def _t5_body(bp_ref, u_ref, dis_ref, w4_ref, b4_ref, h1_ref, h2_ref, h3_ref,
             batch_ref, hcat_ref, key_ref, cnt_ref, sta_ref):
    dis = dis_ref[...]
    sfin = dis * (bp_ref[0, :N, :] + bp_ref[1, :N, :] + u_ref[...])
    h4 = jnp.tanh(jnp.dot(sfin, w4_ref[...], preferred_element_type=jnp.float32)
                  + b4_ref[...])
    hc = jnp.concatenate(
        [h1_ref[...], h2_ref[...], h3_ref[...], h4,
         jnp.zeros((N, 15), jnp.float32)], axis=1)
    hcat_ref[...] = jnp.concatenate(
        [hc, jnp.zeros((NPAD - N, 112), jnp.float32)], axis=0)

    key_ref[...] = jnp.concatenate(
        [h4, jnp.zeros((NPAD - N, 1), jnp.float32)], axis=0)

    gid = lax.broadcasted_iota(jnp.int32, (1, G), 1)
    cmp = (batch_ref[...] == gid).astype(jnp.float32)
    cnt_f = jnp.sum(cmp, axis=0, keepdims=True)
    tri = (lax.broadcasted_iota(jnp.int32, (G, G), 0)
           < lax.broadcasted_iota(jnp.int32, (G, G), 1)).astype(jnp.float32)
    sta_f = jnp.dot(cnt_f, tri, preferred_element_type=jnp.float32)
    cnt_ref[...] = cnt_f.astype(jnp.int32)
    sta_ref[...] = sta_f.astype(jnp.int32)


def _t6_body(pooled_ref, w1c_ref, b1c_ref, wc2_ref, b2c_ref,
             l1w_ref, l1b_ref, l2w_ref, l2b_ref, out_ref):
    pr = pooled_ref[...]
    c1 = jnp.maximum(
        jnp.dot(pr, w1c_ref[...], preferred_element_type=jnp.float32)
        + b1c_ref[...], 0.0)
    c3 = c1.reshape(G, 16, 2, 16)
    p = jnp.max(c3, axis=2)[:, :15, :]
    c2s = []
    for j in range(11):
        win = jnp.concatenate([p[:, j + t, :] for t in range(5)], axis=1)
        c2s.append(jnp.maximum(
            jnp.dot(win, wc2_ref[...], preferred_element_type=jnp.float32)
            + b2c_ref[...], 0.0))
    f = jnp.concatenate([c[:, :, None] for c in c2s], axis=2).reshape(G, 352)
    f = jnp.maximum(
        jnp.dot(f, l1w_ref[...], preferred_element_type=jnp.float32)
        + l1b_ref[...], 0.0)
    z = jnp.dot(f, l2w_ref[...], preferred_element_type=jnp.float32) + l2b_ref[...]
    out_ref[...] = jax.nn.sigmoid(z)


def _tc(body, out_shapes):
    return pl.pallas_call(
        body, out_shape=out_shapes,
        compiler_params=pltpu.CompilerParams(vmem_limit_bytes=100 * 1024 * 1024))


_sc_count = _make_sc_seg(16, gather=False)
_sc_seg32 = _make_sc_seg(32, gather=True)
_sc_pool = _make_sc_pool_topk()

_t1 = _tc(_t1_body, [jax.ShapeDtypeStruct((N, 1), jnp.float32),
                     jax.ShapeDtypeStruct((N, 32), jnp.float32)])
_t2 = _tc(_t2_body, [jax.ShapeDtypeStruct((N, 32), jnp.float32),
                     jax.ShapeDtypeStruct((N, 32), jnp.float32)])
_t5 = _tc(_t5_body, [jax.ShapeDtypeStruct((NPAD, 112), jnp.float32),
                     jax.ShapeDtypeStruct((NPAD, 1), jnp.float32),
                     jax.ShapeDtypeStruct((1, G), jnp.int32),
                     jax.ShapeDtypeStruct((1, G), jnp.int32)])
_t6 = _tc(_t6_body, [jax.ShapeDtypeStruct((G, 1), jnp.float32)])


def kernel(x, edge_index, batch, W1, b1, W2, b2, W3, b3, W4, b4,
           conv1_w, conv1_b, conv2_w, conv2_b, lin1_w, lin1_b, lin2_w, lin2_b):
    src = edge_index[0]
    dst = edge_index[1]

    cnt = _sc_count(src, dst)
    dis, g1 = _t1(cnt, x, W1)
    a1 = _sc_seg32(g1, src, dst)
    h1, g2 = _t2(a1, g1, dis, b1.reshape(1, 32), W2)
    a2 = _sc_seg32(g2, src, dst)
    h2, g3 = _t2(a2, g2, dis, b2.reshape(1, 32), W3)
    a3 = _sc_seg32(g3, src, dst)
    h3, u = _t2(a3, g3, dis, b3.reshape(1, 32), jnp.eye(32, dtype=jnp.float32))
    bb = _sc_seg32(u, src, dst)
    hcat, keyp, cntg, stag = _t5(bb, u, dis, W4, b4.reshape(1, 1), h1, h2, h3,
                                 batch.reshape(N, 1))
    pooled = _sc_pool(hcat, keyp.reshape(NPAD), cntg.reshape(G),
                      stag.reshape(G))

    w1c = jnp.concatenate(
        [conv1_w[:, 0, :].T, jnp.zeros((15, 16), jnp.float32)], axis=0)
    wc2 = conv2_w.transpose(2, 1, 0).reshape(80, 32)
    [out] = _t6(pooled, w1c, conv1_b.reshape(1, 16), wc2,
                conv2_b.reshape(1, 32), lin1_w, lin1_b.reshape(1, 128),
                lin2_w, lin2_b.reshape(1, 1))
    return out


# speedup vs baseline: 1.3117x; 1.0293x over previous
import functools

import jax
import jax.numpy as jnp
from jax import lax
from jax.experimental import pallas as pl
from jax.experimental.pallas import tpu as pltpu
from jax.experimental.pallas import tpu_sc as plsc

N = 10000
E = 320000
G = 64
K = 30
NC = 2
NS = 16
NW = NC * NS
EP = E // NW
CH = 128
NCHF = EP // CH
TAIL = EP - NCHF * CH
NACC = 10240
ZR = NACC // NS
NEG = -1e30
NPAD = N + 16
BPOOL = G * 32


NBUF = 6


def _make_sc_seg(F, gather):
    mesh = plsc.VectorSubcoreMesh(core_axis_name="c", subcore_axis_name="s")
    scratch = [
        pltpu.VMEM((EP,), jnp.int32),
        pltpu.VMEM((EP,), jnp.int32),
        pltpu.VMEM((1, CH), jnp.int32),
        pltpu.VMEM((1, TAIL), jnp.int32),
        pltpu.VMEM((NBUF, CH, F), jnp.float32),
        pltpu.VMEM((TAIL, F), jnp.float32),
        pltpu.VMEM((ZR, F), jnp.float32),
        pltpu.VMEM_SHARED((NACC, F), jnp.float32),
    ] + [pltpu.SemaphoreType.DMA] * (NBUF + 1)
    out_type = jax.ShapeDtypeStruct((NC, NACC, F), jnp.float32)

    def body(*refs):
        if gather:
            (table, ei_h, out, srca, dsta, dstv, dstv2, rows, trows,
             zero_v, acc, *gsems) = refs
        else:
            (ei_h, out, srca, dsta, dstv, dstv2, rows, trows,
             zero_v, acc, *gsems) = refs
        c = lax.axis_index("c")
        s = lax.axis_index("s")
        wid = s * NC + c
        ebase = wid * EP

        pltpu.sync_copy(ei_h.at[1, pl.ds(ebase, EP)], dsta)
        if gather:
            pltpu.sync_copy(ei_h.at[0, pl.ds(ebase, EP)], srca)

        zvec = jnp.zeros((16,), jnp.float32)

        def zrow(i, carry):
            for j in range(F // 16):
                zero_v[i, pl.ds(j * 16, 16)] = zvec
            return carry

        lax.fori_loop(0, ZR, zrow, 0)
        pltpu.sync_copy(zero_v, acc.at[pl.ds(s * ZR, ZR)])
        if gather:
            for b in range(NBUF):
                pltpu.async_copy(table.at[srca.at[pl.ds(b * CH, CH)]],
                                 rows.at[b], gsems[b])
            pltpu.async_copy(table.at[srca.at[pl.ds(NCHF * CH, TAIL)]],
                             trows, gsems[NBUF])
        else:
            ovec = jnp.ones((16,), jnp.float32)

            def orow(i, carry):
                for j in range(F // 16):
                    rows[0, i, pl.ds(j * 16, 16)] = ovec
                return carry

            lax.fori_loop(0, CH, orow, 0)

            def otrow(i, carry):
                for j in range(F // 16):
                    trows[i, pl.ds(j * 16, 16)] = ovec
                return carry

            lax.fori_loop(0, TAIL, otrow, 0)
        plsc.subcore_barrier()

        def group(gidx, carry):
            for b in range(NBUF):
                i = gidx * NBUF + b
                for j in range(CH // 16):
                    dstv[0, pl.ds(j * 16, 16)] = dsta[pl.ds(i * CH + j * 16, 16)]
                if gather:
                    pltpu.make_async_copy(table.at[pl.ds(0, CH)], rows.at[b],
                                          gsems[b]).wait()
                    pltpu.sync_copy(rows.at[b], acc.at[dstv.at[0]], add=True)

                    @pl.when(i + NBUF < NCHF)
                    def _():
                        pltpu.async_copy(
                            table.at[srca.at[pl.ds((i + NBUF) * CH, CH)]],
                            rows.at[b], gsems[b])
                else:
                    pltpu.sync_copy(rows.at[0], acc.at[dstv.at[0]], add=True)
            return carry

        lax.fori_loop(0, NCHF // NBUF, group, 0)
        for j in range(TAIL // 16):
            dstv2[0, pl.ds(j * 16, 16)] = dsta[pl.ds(NCHF * CH + j * 16, 16)]
        if gather:
            pltpu.make_async_copy(table.at[pl.ds(0, TAIL)], trows,
                                  gsems[NBUF]).wait()
        pltpu.sync_copy(trows, acc.at[dstv2.at[0]], add=True)
        plsc.subcore_barrier()
        pltpu.sync_copy(acc.at[pl.ds(s * ZR, ZR)], out.at[c, pl.ds(s * ZR, ZR)])

    return pl.kernel(body, out_type=out_type, mesh=mesh, scratch_types=scratch,
                     compiler_params=pltpu.CompilerParams(use_tc_tiling_on_sc=False))


def _make_sc_pool_topk():
    mesh = plsc.VectorSubcoreMesh(core_axis_name="c", subcore_axis_name="s")
    per_w = BPOOL // NW
    scratch = [
        pltpu.VMEM((NPAD,), jnp.float32),
        pltpu.VMEM((G + 16,), jnp.int32),
        pltpu.VMEM((G + 16,), jnp.int32),
        pltpu.VMEM((1, per_w), jnp.int32),
        pltpu.VMEM((per_w, 112), jnp.float32),
        pltpu.SemaphoreType.DMA,
    ]
    out_type = jax.ShapeDtypeStruct((BPOOL, 112), jnp.float32)
    LOW = -3e38

    def body(table, key_h, cnt_h, sta_h, out, key_v, cnt_v, sta_v,
             idx_v, rows_v, sem):
        c = lax.axis_index("c")
        s = lax.axis_index("s")
        wid = s * NC + c
        pltpu.sync_copy(key_h, key_v)
        pltpu.sync_copy(cnt_h, cnt_v.at[pl.ds(0, G)])
        pltpu.sync_copy(sta_h, sta_v.at[pl.ds(0, G)])
        lane = lax.iota(jnp.int32, 16)
        sent = jnp.full((16,), N, jnp.int32)
        for h in range(per_w // 16):
            idx_v[0, pl.ds(h * 16, 16)] = sent
        for t in range(2):
            g = wid * 2 + t
            cnt = cnt_v[pl.ds(g, 16)][0]
            start = sta_v[pl.ds(g, 16)][0]
            nv = (cnt + 15) >> 4

            def select(k, kcarry):
                vt, li = kcarry

                def scan(j, carry):
                    bv, bp = carry
                    v = key_v[pl.ds(start + j * 16, 16)]
                    pos = lane + j * 16
                    ok = ((pos < cnt)
                          & ((v < vt) | ((v == vt) & (pos > li))))
                    vm = jnp.where(ok, v, LOW)
                    m = jnp.max(vm)
                    p = jnp.min(jnp.where(vm == m, pos, N))
                    better = m > bv
                    return (jnp.where(better, m, bv),
                            jnp.where(better, p, bp))

                bv, bp = lax.fori_loop(0, nv, scan,
                                       (jnp.float32(LOW), jnp.int32(N)))
                gi = jnp.where(bv > jnp.float32(-1e30), start + bp, N)
                slot = t * 32 + k
                hb = (slot >> 4) << 4
                lane_in = slot & 15
                cur = idx_v[0, pl.ds(hb, 16)]
                idx_v[0, pl.ds(hb, 16)] = jnp.where(lane == lane_in, gi, cur)
                return bv, bp

            lax.fori_loop(0, K, select, (jnp.float32(3e38), jnp.int32(-1)))
        pltpu.async_copy(table.at[idx_v.at[0]], rows_v, sem).wait()
        pltpu.sync_copy(rows_v, out.at[pl.ds(wid * per_w, per_w)])

    return pl.kernel(body, out_type=out_type, mesh=mesh, scratch_types=scratch,
                     compiler_params=pltpu.CompilerParams(
                         use_tc_tiling_on_sc=False, needs_layout_passes=False))


def _t1_body(cnt_ref, x_ref, w1_ref, dis_ref, g1_ref):
    deg = cnt_ref[0, :N, 0:1] + cnt_ref[1, :N, 0:1] + 1.0
    dis = lax.rsqrt(deg)
    m = jnp.dot(x_ref[...], w1_ref[...], preferred_element_type=jnp.float32)
    dis_ref[...] = dis
    g1_ref[...] = dis * m


def _t2_body(ap_ref, g_ref, dis_ref, b_ref, w_ref, h_ref, gn_ref):
    dis = dis_ref[...]
    h = jnp.tanh(dis * (ap_ref[0, :N, :] + ap_ref[1, :N, :] + g_ref[...]) + b_ref[...])
    h_ref[...] = h
    gn_ref[...] = dis * jnp.dot(h, w_ref[...], preferred_element_type=jnp.float32)


def _t5_body(bp_ref, u_ref, dis_ref, w4_ref, b4_ref, h1_ref, h2_ref, h3_ref,
             batch_ref, hcat_ref, key_ref, cnt_ref, sta_ref):
    dis = dis_ref[...]
    sfin = dis * (bp_ref[0, :N, :] + bp_ref[1, :N, :] + u_ref[...])
    h4 = jnp.tanh(jnp.dot(sfin, w4_ref[...], preferred_element_type=jnp.float32)
                  + b4_ref[...])
    hc = jnp.concatenate(
        [h1_ref[...], h2_ref[...], h3_ref[...], h4,
         jnp.zeros((N, 15), jnp.float32)], axis=1)
    hcat_ref[...] = jnp.concatenate(
        [hc, jnp.zeros((NPAD - N, 112), jnp.float32)], axis=0)

    key_ref[...] = jnp.concatenate(
        [h4, jnp.zeros((NPAD - N, 1), jnp.float32)], axis=0)

    gid = lax.broadcasted_iota(jnp.int32, (1, G), 1)
    cmp = (batch_ref[...] == gid).astype(jnp.float32)
    cnt_f = jnp.sum(cmp, axis=0, keepdims=True)
    tri = (lax.broadcasted_iota(jnp.int32, (G, G), 0)
           < lax.broadcasted_iota(jnp.int32, (G, G), 1)).astype(jnp.float32)
    sta_f = jnp.dot(cnt_f, tri, preferred_element_type=jnp.float32)
    cnt_ref[...] = cnt_f.astype(jnp.int32)
    sta_ref[...] = sta_f.astype(jnp.int32)


def _t6_body(pooled_ref, w1c_ref, b1c_ref, wc2_ref, b2c_ref,
             l1w_ref, l1b_ref, l2w_ref, l2b_ref, out_ref):
    pr = pooled_ref[...]
    c1 = jnp.maximum(
        jnp.dot(pr, w1c_ref[...], preferred_element_type=jnp.float32)
        + b1c_ref[...], 0.0)
    c3 = c1.reshape(G, 16, 2, 16)
    p = jnp.max(c3, axis=2)[:, :15, :]
    c2s = []
    for j in range(11):
        win = jnp.concatenate([p[:, j + t, :] for t in range(5)], axis=1)
        c2s.append(jnp.maximum(
            jnp.dot(win, wc2_ref[...], preferred_element_type=jnp.float32)
            + b2c_ref[...], 0.0))
    f = jnp.concatenate([c[:, :, None] for c in c2s], axis=2).reshape(G, 352)
    f = jnp.maximum(
        jnp.dot(f, l1w_ref[...], preferred_element_type=jnp.float32)
        + l1b_ref[...], 0.0)
    z = jnp.dot(f, l2w_ref[...], preferred_element_type=jnp.float32) + l2b_ref[...]
    out_ref[...] = jax.nn.sigmoid(z)


def _tc(body, out_shapes):
    return pl.pallas_call(
        body, out_shape=out_shapes,
        compiler_params=pltpu.CompilerParams(vmem_limit_bytes=100 * 1024 * 1024))


_sc_count = _make_sc_seg(16, gather=False)
_sc_seg32 = _make_sc_seg(32, gather=True)
_sc_pool = _make_sc_pool_topk()

_t1 = _tc(_t1_body, [jax.ShapeDtypeStruct((N, 1), jnp.float32),
                     jax.ShapeDtypeStruct((N, 32), jnp.float32)])
_t2 = _tc(_t2_body, [jax.ShapeDtypeStruct((N, 32), jnp.float32),
                     jax.ShapeDtypeStruct((N, 32), jnp.float32)])
_t5 = _tc(_t5_body, [jax.ShapeDtypeStruct((NPAD, 112), jnp.float32),
                     jax.ShapeDtypeStruct((NPAD, 1), jnp.float32),
                     jax.ShapeDtypeStruct((1, G), jnp.int32),
                     jax.ShapeDtypeStruct((1, G), jnp.int32)])
_t6 = _tc(_t6_body, [jax.ShapeDtypeStruct((G, 1), jnp.float32)])


def kernel(x, edge_index, batch, W1, b1, W2, b2, W3, b3, W4, b4,
           conv1_w, conv1_b, conv2_w, conv2_b, lin1_w, lin1_b, lin2_w, lin2_b):
    cnt = _sc_count(edge_index)
    dis, g1 = _t1(cnt, x, W1)
    a1 = _sc_seg32(g1, edge_index)
    h1, g2 = _t2(a1, g1, dis, b1.reshape(1, 32), W2)
    a2 = _sc_seg32(g2, edge_index)
    h2, g3 = _t2(a2, g2, dis, b2.reshape(1, 32), W3)
    a3 = _sc_seg32(g3, edge_index)
    h3, u = _t2(a3, g3, dis, b3.reshape(1, 32), jnp.eye(32, dtype=jnp.float32))
    bb = _sc_seg32(u, edge_index)
    hcat, keyp, cntg, stag = _t5(bb, u, dis, W4, b4.reshape(1, 1), h1, h2, h3,
                                 batch.reshape(N, 1))
    pooled = _sc_pool(hcat, keyp.reshape(NPAD), cntg.reshape(G),
                      stag.reshape(G))

    w1c = jnp.concatenate(
        [conv1_w[:, 0, :].T, jnp.zeros((15, 16), jnp.float32)], axis=0)
    wc2 = conv2_w.transpose(2, 1, 0).reshape(80, 32)
    [out] = _t6(pooled, w1c, conv1_b.reshape(1, 16), wc2,
                conv2_b.reshape(1, 32), lin1_w, lin1_b.reshape(1, 128),
                lin2_w, lin2_b.reshape(1, 1))
    return out
